# Initial kernel scaffold; baseline (speedup 1.0000x reference)
import functools

import jax
import jax.numpy as jnp
from jax import lax
from jax.experimental import pallas as pl
from jax.experimental.pallas import tpu as pltpu
from jax.experimental.pallas import tpu_sc as plsc

N = 10000
E = 320000
LAG = 50
FEA = 4
HID = 128
G4 = 4 * HID

NB = 1000
NC, NS = 2, 16
CH = 80
RPT = N // NS

_PREC_HI = lax.Precision.HIGHEST

_vec_mesh = plsc.VectorSubcoreMesh(core_axis_name="c", subcore_axis_name="s")


def _lstm_body(yx_ref, wih_ref, whhT_ref, b_ref, wt_ref, wfc_ref, bfc_ref,
               out_ref, h_ref, c_ref, acc_ref, den_ref):
    h_ref[...] = jnp.zeros_like(h_ref)
    c_ref[...] = jnp.zeros_like(c_ref)
    acc_ref[...] = jnp.zeros_like(acc_ref)
    den_ref[...] = jnp.zeros_like(den_ref)
    yx = yx_ref[...]
    wih = wih_ref[...]
    whhT = whhT_ref[...]
    b = b_ref[...]
    wt_all = wt_ref[...]

    def step(t, carry):
        oh_col = (lax.broadcasted_iota(jnp.int32, (LAG, 1), 0) == t)
        xt = jnp.dot(yx, oh_col.astype(jnp.float32),
                     preferred_element_type=jnp.float32)
        gates = (jnp.dot(h_ref[...], whhT,
                         preferred_element_type=jnp.float32,
                         precision=_PREC_HI)
                 + xt * wih + b)
        i = jax.nn.sigmoid(gates[:, :HID])
        f = jax.nn.sigmoid(gates[:, HID:2 * HID])
        g = jnp.tanh(gates[:, 2 * HID:3 * HID])
        o = jax.nn.sigmoid(gates[:, 3 * HID:])
        c = f * c_ref[...] + i * g
        hn = o * jnp.tanh(c)
        oh_row = (lax.broadcasted_iota(jnp.int32, (1, LAG), 1) == t)
        wt = jnp.dot(oh_row.astype(jnp.float32), wt_all,
                     preferred_element_type=jnp.float32)
        s = jnp.sum(hn * wt, axis=1, keepdims=True)
        e = jnp.exp(s)
        acc_ref[...] = acc_ref[...] + e * hn
        den_ref[...] = den_ref[...] + jnp.broadcast_to(e, den_ref.shape)
        h_ref[...] = hn
        c_ref[...] = c
        return carry

    lax.fori_loop(0, LAG, step, 0)
    att = acc_ref[...] / den_ref[...]
    out_ref[...] = jnp.maximum(
        jnp.dot(att, wfc_ref[...], preferred_element_type=jnp.float32,
                precision=_PREC_HI) + bfc_ref[...], 0.0)


def _tc_lstm(node_yx, wih_row, whhT, bsum, W_t, W_fc, bfc_row):
    f32 = jnp.float32
    return pl.pallas_call(
        _lstm_body,
        grid=(N // NB,),
        in_specs=[
            pl.BlockSpec((NB, LAG), lambda i: (i, 0)),
            pl.BlockSpec((1, G4), lambda i: (0, 0)),
            pl.BlockSpec((HID, G4), lambda i: (0, 0)),
            pl.BlockSpec((1, G4), lambda i: (0, 0)),
            pl.BlockSpec((LAG, HID), lambda i: (0, 0)),
            pl.BlockSpec((HID, HID), lambda i: (0, 0)),
            pl.BlockSpec((1, HID), lambda i: (0, 0)),
        ],
        out_specs=pl.BlockSpec((NB, HID), lambda i: (i, 0)),
        out_shape=jax.ShapeDtypeStruct((N, HID), f32),
        scratch_shapes=[pltpu.VMEM((NB, HID), f32)] * 4,
    )(node_yx, wih_row, whhT, bsum, W_t, W_fc, bfc_row)


def _sc_deg(col, ew):
    f32 = jnp.float32
    EPC = E // NC
    EPT = EPC // NS
    NCHUNK = EPT // CH

    @functools.partial(
        pl.kernel,
        out_type=jax.ShapeDtypeStruct((NC, N, 16), f32),
        mesh=_vec_mesh,
        scratch_types=[
            pltpu.VMEM((1, CH), jnp.int32),
            pltpu.VMEM((CH,), f32),
            pltpu.VMEM((CH, 16), f32),
            pltpu.VMEM((RPT, 16), f32),
            pltpu.VMEM_SHARED((N, 16), f32),
        ],
    )
    def k(col_hbm, ew_hbm, out_hbm, colv, ewv, pay, zbuf, acc):
        c = lax.axis_index("c")
        s = lax.axis_index("s")
        zero16 = jnp.zeros((16,), f32)

        @pl.loop(0, RPT)
        def _(i):
            zbuf[i, :] = zero16

        pltpu.sync_copy(zbuf, acc.at[pl.ds(s * RPT, RPT)])

        @pl.loop(0, CH)
        def _(i):
            pay[i, :] = zero16

        plsc.subcore_barrier()
        base = c * EPC + s * EPT
        iota16 = lax.broadcasted_iota(jnp.int32, (16,), 0)
        zero_i = jnp.zeros((16,), jnp.int32)

        @pl.loop(0, NCHUNK)
        def _(kk):
            off = base + kk * CH
            pltpu.sync_copy(col_hbm.at[pl.ds(off, CH)], colv.at[0])
            pltpu.sync_copy(ew_hbm.at[pl.ds(off, CH)], ewv)

            @pl.loop(0, CH, step=16)
            def _(i):
                ew16 = ewv[pl.ds(i, 16)]
                plsc.store_scatter(pay, [i + iota16, zero_i], ew16)

            pltpu.sync_copy(pay, acc.at[colv.at[0]], add=True)

        plsc.subcore_barrier()
        pltpu.sync_copy(acc.at[pl.ds(s * RPT, RPT)],
                        out_hbm.at[c].at[pl.ds(s * RPT, RPT)])

    return k(col, ew)


def _sc_propagate(yI, yV, row, col, ew):
    f32 = jnp.float32
    EPT = E // NS
    NCHUNK = EPT // CH

    @functools.partial(
        pl.kernel,
        out_type=(jax.ShapeDtypeStruct((N, HID), f32),
                  jax.ShapeDtypeStruct((N, HID), f32)),
        mesh=_vec_mesh,
        scratch_types=[
            pltpu.VMEM((CH,), jnp.int32),
            pltpu.VMEM((1, CH), jnp.int32),
            pltpu.VMEM((CH,), f32),
            pltpu.VMEM((CH, HID), f32),
            pltpu.VMEM((125, HID), f32),
            pltpu.VMEM_SHARED((N, HID), f32),
            pltpu.SemaphoreType.DMA,
        ],
    )
    def k(yI_hbm, yV_hbm, row_hbm, col_hbm, ew_hbm, outI_hbm, outV_hbm,
          rowv, colv, ewv, gbuf, zbuf, acc, sem):
        c = lax.axis_index("c")
        s = lax.axis_index("s")
        zero16 = jnp.zeros((16,), f32)

        @pl.loop(0, 125)
        def _(i):
            for j in range(HID // 16):
                zbuf[i, pl.ds(j * 16, 16)] = zero16

        @pl.loop(0, 5)
        def _(j):
            pltpu.sync_copy(zbuf, acc.at[pl.ds(s * RPT + j * 125, 125)])

        plsc.subcore_barrier()
        base = s * EPT

        @pl.loop(0, NCHUNK)
        def _(kk):
            off = base + kk * CH
            pltpu.sync_copy(row_hbm.at[pl.ds(off, CH)], rowv)
            pltpu.sync_copy(col_hbm.at[pl.ds(off, CH)], colv.at[0])
            pltpu.sync_copy(ew_hbm.at[pl.ds(off, CH)], ewv)

            @pl.when(c == 0)
            def _():
                pltpu.async_copy(yI_hbm.at[rowv], gbuf, sem).wait()

            @pl.when(c == 1)
            def _():
                pltpu.async_copy(yV_hbm.at[rowv], gbuf, sem).wait()

            @pl.loop(0, CH)
            def _(e):
                cv = lax.broadcast_in_dim(ewv[e], (16,), ())
                for j in range(HID // 16):
                    sl = pl.ds(j * 16, 16)
                    gbuf[e, sl] = gbuf[e, sl] * cv

            pltpu.sync_copy(gbuf, acc.at[colv.at[0]], add=True)

        plsc.subcore_barrier()

        @pl.when(c == 0)
        def _():
            pltpu.sync_copy(acc.at[pl.ds(s * RPT, RPT)],
                            outI_hbm.at[pl.ds(s * RPT, RPT)])

        @pl.when(c == 1)
        def _():
            pltpu.sync_copy(acc.at[pl.ds(s * RPT, RPT)],
                            outV_hbm.at[pl.ds(s * RPT, RPT)])

    return k(yI, yV, row, col, ew)


def _sc_vstar(su, v, perm):
    f32 = jnp.float32
    TPT = 400

    @functools.partial(
        pl.kernel,
        out_type=jax.ShapeDtypeStruct((N,), f32),
        mesh=_vec_mesh,
        scratch_types=[
            pltpu.VMEM((N,), f32),
            pltpu.VMEM((TPT,), f32),
            pltpu.VMEM((TPT,), jnp.int32),
            pltpu.VMEM((TPT,), f32),
        ],
    )
    def k(su_hbm, v_hbm, perm_hbm, out_hbm, vv, suv, pv, ov):
        c = lax.axis_index("c")
        s = lax.axis_index("s")
        wid = s * NC + c

        @pl.when(wid < N // TPT)
        def _():
            base = wid * TPT
            pltpu.sync_copy(v_hbm, vv)
            pltpu.sync_copy(su_hbm.at[pl.ds(base, TPT)], suv)
            pltpu.sync_copy(perm_hbm.at[pl.ds(base, TPT)], pv)

            @pl.loop(0, TPT, step=16)
            def _(i):
                idx16 = pv[pl.ds(i, 16)]
                vg = plsc.load_gather(vv, [idx16])
                ov[pl.ds(i, 16)] = jnp.maximum(suv[pl.ds(i, 16)] + vg, 0.0)

            pltpu.sync_copy(ov, out_hbm.at[pl.ds(base, TPT)])

    return k(su, v, perm)


def _prep_body(dp_ref, x1_ref, nx_ref, wia_ref, wib_ref, wva_ref, wvb_ref,
               dinv_ref, yI_ref, yV_ref):
    dp = dp_ref[...]
    deg = jnp.sum(dp[0], axis=1) + jnp.sum(dp[1], axis=1) + 1.0
    dinv = lax.rsqrt(deg)[:, None]
    dinv_ref[...] = jnp.broadcast_to(dinv, dinv_ref.shape)
    x1b = x1_ref[...]
    nxb = nx_ref[...]
    xwI = (jnp.dot(x1b, wia_ref[...], preferred_element_type=jnp.float32,
                   precision=_PREC_HI)
           + jnp.dot(nxb, wib_ref[...], preferred_element_type=jnp.float32,
                     precision=_PREC_HI))
    xwV = (jnp.dot(x1b, wva_ref[...], preferred_element_type=jnp.float32,
                   precision=_PREC_HI)
           + jnp.dot(nxb, wvb_ref[...], preferred_element_type=jnp.float32,
                     precision=_PREC_HI))
    yI_ref[...] = dinv * xwI
    yV_ref[...] = dinv * xwV


def _tc_prep(degp, x1, node_x, wia, wib, wva, wvb):
    f32 = jnp.float32
    full = lambda a, b: pl.BlockSpec((a, b), lambda i: (0, 0))
    return pl.pallas_call(
        _prep_body,
        grid=(N // NB,),
        in_specs=[
            pl.BlockSpec((NC, NB, 16), lambda i: (0, i, 0)),
            pl.BlockSpec((NB, HID), lambda i: (i, 0)),
            pl.BlockSpec((NB, FEA), lambda i: (i, 0)),
            full(HID, HID), full(FEA, HID), full(HID, HID), full(FEA, HID),
        ],
        out_specs=[pl.BlockSpec((NB, HID), lambda i: (i, 0))] * 3,
        out_shape=[jax.ShapeDtypeStruct((N, HID), f32)] * 3,
    )(degp, x1, node_x, wia, wib, wva, wvb)


def _mid_body(sI_ref, sV_ref, yI_ref, yV_ref, dinv_ref, bI_ref, bV_ref,
              wI2_ref, wV2_ref, y2I_ref, y2V_ref):
    dinv = dinv_ref[...]
    tI = dinv * (sI_ref[...] + yI_ref[...]) + bI_ref[...]
    tV = dinv * (sV_ref[...] + yV_ref[...]) + bV_ref[...]
    y2I_ref[...] = dinv * jnp.dot(tI, wI2_ref[...],
                                  preferred_element_type=jnp.float32,
                                  precision=_PREC_HI)
    y2V_ref[...] = dinv * jnp.dot(tV, wV2_ref[...],
                                  preferred_element_type=jnp.float32,
                                  precision=_PREC_HI)


def _tc_mid(sI, sV, yI, yV, dinvb, bI1, bV1, wI2, wV2):
    f32 = jnp.float32
    blk = pl.BlockSpec((NB, HID), lambda i: (i, 0))
    full = lambda a, b: pl.BlockSpec((a, b), lambda i: (0, 0))
    return pl.pallas_call(
        _mid_body,
        grid=(N // NB,),
        in_specs=[blk, blk, blk, blk, blk,
                  full(1, HID), full(1, HID), full(HID, HID), full(HID, HID)],
        out_specs=[blk] * 2,
        out_shape=[jax.ShapeDtypeStruct((N, HID), f32)] * 2,
    )(sI, sV, yI, yV, dinvb, bI1, bV1, wI2, wV2)


def _fin_body(s2I_ref, s2V_ref, y2I_ref, y2V_ref, dinv_ref, bI_ref, bV_ref,
              wlI_ref, wlV_ref, bl_ref,
              x2I_ref, x2V_ref, pred_ref, su_ref, v_ref):
    dinv = dinv_ref[...]
    x2I = dinv * (s2I_ref[...] + y2I_ref[...]) + bI_ref[...]
    x2V = dinv * (s2V_ref[...] + y2V_ref[...]) + bV_ref[...]
    x2I_ref[...] = x2I
    x2V_ref[...] = x2V
    u = jnp.dot(x2I, wlI_ref[...], preferred_element_type=jnp.float32,
                precision=_PREC_HI)
    v = jnp.dot(x2V, wlV_ref[...], preferred_element_type=jnp.float32,
                precision=_PREC_HI)
    bl = bl_ref[...]
    pred_ref[...] = jnp.maximum(u + v + bl, 0.0)
    su_ref[...] = u + bl
    v_ref[...] = v


def _tc_fin(s2I, s2V, y2I, y2V, dinvb, bI2, bV2, wlI, wlV, blin):
    f32 = jnp.float32
    blk = pl.BlockSpec((NB, HID), lambda i: (i, 0))
    col = pl.BlockSpec((NB, 1), lambda i: (i, 0))
    full = lambda a, b: pl.BlockSpec((a, b), lambda i: (0, 0))
    return pl.pallas_call(
        _fin_body,
        grid=(N // NB,),
        in_specs=[blk, blk, blk, blk, blk,
                  full(1, HID), full(1, HID),
                  full(HID, 1), full(HID, 1), full(1, 1)],
        out_specs=[blk, blk, col, col, col],
        out_shape=[jax.ShapeDtypeStruct((N, HID), f32)] * 2
        + [jax.ShapeDtypeStruct((N, 1), f32)] * 3,
    )(s2I, s2V, y2I, y2V, dinvb, bI2, bV2, wlI, wlV, blin)


def kernel(node_x, node_yx, edge_index, edge_weight,
           W_ih, W_hh, b_ih, b_hh, W_t, W_fc, b_fc,
           W_I1, b_I1, W_I2, b_I2, W_V1, b_V1, W_V2, b_V2,
           W_lin, b_lin, perm):
    row = edge_index[0]
    col = edge_index[1]
    perm32 = perm.astype(jnp.int32)

    degp = _sc_deg(col, edge_weight)
    x1 = _tc_lstm(node_yx, W_ih.T, W_hh.T, (b_ih + b_hh)[None, :],
                  W_t, W_fc, b_fc[None, :])
    dinvb, yI, yV = _tc_prep(degp, x1, node_x,
                             W_I1[:HID], W_I1[HID:], W_V1[:HID], W_V1[HID:])
    sI, sV = _sc_propagate(yI, yV, row, col, edge_weight)
    y2I, y2V = _tc_mid(sI, sV, yI, yV, dinvb,
                       b_I1[None, :], b_V1[None, :], W_I2, W_V2)
    s2I, s2V = _sc_propagate(y2I, y2V, row, col, edge_weight)
    x2I, x2V, pred, su, v = _tc_fin(s2I, s2V, y2I, y2V, dinvb,
                                    b_I2[None, :], b_V2[None, :],
                                    W_lin[:HID], W_lin[HID:],
                                    b_lin[None, :])
    pred_star = _sc_vstar(su.reshape(N), v.reshape(N), perm32)
    return (pred.reshape(N), pred_star, x2I, x2V)


# speedup vs baseline: 4.6657x; 4.6657x over previous
import dataclasses
import functools

import jax
import jax.numpy as jnp
from jax import lax
from jax.experimental import pallas as pl
from jax.experimental.pallas import tpu as pltpu
from jax.experimental.pallas import tpu_sc as plsc

N = 10000
E = 320000
LAG = 50
FEA = 4
HID = 128
G4 = 4 * HID

NB = 1000
NC, NS = 2, 16
CH = 80
RPT = N // NS
RSTRIDE = 624
RSPAN = 640

_PREC_HI = lax.Precision.HIGHEST

@functools.cache
def _vec_mesh():
    return plsc.VectorSubcoreMesh(core_axis_name="c", subcore_axis_name="s",
                                  num_cores=NC, num_subcores=NS)


def _sc_params():
    cp = pltpu.CompilerParams()
    if "needs_layout_passes" in pltpu.CompilerParams.__dataclass_fields__:
        cp = dataclasses.replace(cp, needs_layout_passes=False)
    return cp


def _lstm_body(yx_ref, wih_ref, whhT_ref, b_ref, wt_ref, wfc_ref, bfc_ref,
               out_ref, h_ref, c_ref, acc_ref, den_ref):
    h_ref[...] = jnp.zeros_like(h_ref)
    c_ref[...] = jnp.zeros_like(c_ref)
    acc_ref[...] = jnp.zeros_like(acc_ref)
    den_ref[...] = jnp.zeros_like(den_ref)
    yx = yx_ref[...]
    wih = wih_ref[...]
    whhT = whhT_ref[...]
    b = b_ref[...]
    wt_all = wt_ref[...]

    def step(t, carry):
        oh_col = (lax.broadcasted_iota(jnp.int32, (LAG, 1), 0) == t)
        xt = jnp.dot(yx, oh_col.astype(jnp.float32),
                     preferred_element_type=jnp.float32)
        gates = (jnp.dot(h_ref[...], whhT,
                         preferred_element_type=jnp.float32,
                         precision=_PREC_HI)
                 + xt * wih + b)
        i = jax.nn.sigmoid(gates[:, :HID])
        f = jax.nn.sigmoid(gates[:, HID:2 * HID])
        g = jnp.tanh(gates[:, 2 * HID:3 * HID])
        o = jax.nn.sigmoid(gates[:, 3 * HID:])
        c = f * c_ref[...] + i * g
        hn = o * jnp.tanh(c)
        oh_row = (lax.broadcasted_iota(jnp.int32, (1, LAG), 1) == t)
        wt = jnp.dot(oh_row.astype(jnp.float32), wt_all,
                     preferred_element_type=jnp.float32)
        s = jnp.sum(hn * wt, axis=1, keepdims=True)
        e = jnp.exp(s)
        acc_ref[...] = acc_ref[...] + e * hn
        den_ref[...] = den_ref[...] + jnp.broadcast_to(e, den_ref.shape)
        h_ref[...] = hn
        c_ref[...] = c
        return carry

    lax.fori_loop(0, LAG, step, 0)
    att = acc_ref[...] / den_ref[...]
    out_ref[...] = jnp.maximum(
        jnp.dot(att, wfc_ref[...], preferred_element_type=jnp.float32,
                precision=_PREC_HI) + bfc_ref[...], 0.0)


def _tc_lstm(node_yx, wih_row, whhT, bsum, W_t, W_fc, bfc_row):
    f32 = jnp.float32
    return pl.pallas_call(
        _lstm_body,
        grid=(N // NB,),
        in_specs=[
            pl.BlockSpec((NB, LAG), lambda i: (i, 0)),
            pl.BlockSpec((1, G4), lambda i: (0, 0)),
            pl.BlockSpec((HID, G4), lambda i: (0, 0)),
            pl.BlockSpec((1, G4), lambda i: (0, 0)),
            pl.BlockSpec((LAG, HID), lambda i: (0, 0)),
            pl.BlockSpec((HID, HID), lambda i: (0, 0)),
            pl.BlockSpec((1, HID), lambda i: (0, 0)),
        ],
        out_specs=pl.BlockSpec((NB, HID), lambda i: (i, 0)),
        out_shape=jax.ShapeDtypeStruct((N, HID), f32),
        scratch_shapes=[pltpu.VMEM((NB, HID), f32)] * 4,
    )(node_yx, wih_row, whhT, bsum, W_t, W_fc, bfc_row)


def _sc_deg(col, ew):
    f32 = jnp.float32
    NW = NC * NS
    EPW = E // NW
    CHD = 2000
    NCHUNK = EPW // CHD

    @functools.partial(
        pl.kernel,
        out_type=jax.ShapeDtypeStruct((NW, 1, N), f32),
        mesh=_vec_mesh(),
        compiler_params=_sc_params(),
        scratch_types=[
            pltpu.VMEM((N,), f32),
            pltpu.VMEM((CHD,), jnp.int32),
            pltpu.VMEM((CHD,), f32),
        ],
    )
    def k(col_hbm, ew_hbm, out_hbm, dacc, colv, ewv):
        c = lax.axis_index("c")
        s = lax.axis_index("s")
        wid = s * NC + c
        zero16 = jnp.zeros((16,), f32)

        @pl.loop(0, N, step=16)
        def _(i):
            dacc[pl.ds(i, 16)] = zero16

        base = wid * EPW

        @pl.loop(0, NCHUNK)
        def _(kk):
            off = base + kk * CHD
            pltpu.sync_copy(col_hbm.at[pl.ds(off, CHD)], colv)
            pltpu.sync_copy(ew_hbm.at[pl.ds(off, CHD)], ewv)

            @pl.loop(0, CHD, step=16)
            def _(i):
                plsc.addupdate_scatter(dacc, [colv[pl.ds(i, 16)]],
                                       ewv[pl.ds(i, 16)])

        pltpu.sync_copy(dacc, out_hbm.at[wid].at[0])

    return k(col, ew)


def _sc_propagate(yI, yV, row, col, ew):
    f32 = jnp.float32
    EPT = E // NS
    NCHUNK = EPT // CH

    @functools.partial(
        pl.kernel,
        out_type=(jax.ShapeDtypeStruct((N, HID), f32),
                  jax.ShapeDtypeStruct((N, HID), f32)),
        mesh=_vec_mesh(),
        compiler_params=_sc_params(),
        scratch_types=[
            pltpu.VMEM((CH,), jnp.int32),
            pltpu.VMEM((1, CH), jnp.int32),
            pltpu.VMEM((CH,), f32),
            pltpu.VMEM((CH, HID), f32),
            pltpu.VMEM((128, HID), f32),
            pltpu.VMEM_SHARED((N, HID), f32),
            pltpu.SemaphoreType.DMA,
        ],
    )
    def k(yI_hbm, yV_hbm, row_hbm, col_hbm, ew_hbm, outI_hbm, outV_hbm,
          rowv, colv, ewv, gbuf, zbuf, acc, sem):
        c = lax.axis_index("c")
        s = lax.axis_index("s")
        zero16 = jnp.zeros((16,), f32)

        @pl.loop(0, 128)
        def _(i):
            for j in range(HID // 16):
                zbuf[i, pl.ds(j * 16, 16)] = zero16

        @pl.loop(0, 5)
        def _(j):
            pltpu.sync_copy(zbuf, acc.at[pl.ds(s * RSTRIDE + j * 128, 128)])

        plsc.subcore_barrier()
        base = s * EPT

        @pl.loop(0, NCHUNK)
        def _(kk):
            off = base + kk * CH
            pltpu.sync_copy(row_hbm.at[pl.ds(off, CH)], rowv)
            pltpu.sync_copy(col_hbm.at[pl.ds(off, CH)], colv.at[0])
            pltpu.sync_copy(ew_hbm.at[pl.ds(off, CH)], ewv)

            @pl.when(c == 0)
            def _():
                pltpu.async_copy(yI_hbm.at[rowv], gbuf, sem).wait()

            @pl.when(c == 1)
            def _():
                pltpu.async_copy(yV_hbm.at[rowv], gbuf, sem).wait()

            @pl.loop(0, CH, step=16)
            def _(i):
                ew16 = ewv[pl.ds(i, 16)]
                for l in range(16):
                    cv = lax.broadcast_in_dim(ew16[l], (16,), ())
                    for j in range(HID // 16):
                        sl = pl.ds(j * 16, 16)
                        gbuf[i + l, sl] = gbuf[i + l, sl] * cv

            pltpu.sync_copy(gbuf, acc.at[colv.at[0]], add=True)

        plsc.subcore_barrier()

        @pl.when(c == 0)
        def _():
            pltpu.sync_copy(acc.at[pl.ds(s * RSTRIDE, RSPAN)],
                            outI_hbm.at[pl.ds(s * RSTRIDE, RSPAN)])

        @pl.when(c == 1)
        def _():
            pltpu.sync_copy(acc.at[pl.ds(s * RSTRIDE, RSPAN)],
                            outV_hbm.at[pl.ds(s * RSTRIDE, RSPAN)])

    return k(yI, yV, row, col, ew)


def _sc_vstar(su, v, perm):
    f32 = jnp.float32
    TPT = 400

    @functools.partial(
        pl.kernel,
        out_type=jax.ShapeDtypeStruct((N,), f32),
        mesh=_vec_mesh(),
        compiler_params=_sc_params(),
        scratch_types=[
            pltpu.VMEM((N,), f32),
            pltpu.VMEM((TPT,), f32),
            pltpu.VMEM((TPT,), jnp.int32),
            pltpu.VMEM((TPT,), f32),
        ],
    )
    def k(su_hbm, v_hbm, perm_hbm, out_hbm, vv, suv, pv, ov):
        c = lax.axis_index("c")
        s = lax.axis_index("s")
        wid = s * NC + c

        @pl.when(wid < N // TPT)
        def _():
            base = wid * TPT
            pltpu.sync_copy(v_hbm, vv)
            pltpu.sync_copy(su_hbm.at[pl.ds(base, TPT)], suv)
            pltpu.sync_copy(perm_hbm.at[pl.ds(base, TPT)], pv)

            @pl.loop(0, TPT, step=16)
            def _(i):
                idx16 = pv[pl.ds(i, 16)]
                vg = plsc.load_gather(vv, [idx16])
                ov[pl.ds(i, 16)] = jnp.maximum(suv[pl.ds(i, 16)] + vg, 0.0)

            pltpu.sync_copy(ov, out_hbm.at[pl.ds(base, TPT)])

    return k(su, v, perm)


def _prep_body(dp_ref, x1_ref, nx_ref, wia_ref, wib_ref, wva_ref, wvb_ref,
               dinv_ref, yI_ref, yV_ref):
    dp = dp_ref[...]
    deg = jnp.sum(dp[0], axis=0) + 1.0
    dinv = lax.rsqrt(deg)[:, None]
    dinv_ref[...] = jnp.broadcast_to(dinv, dinv_ref.shape)
    x1b = x1_ref[...]
    nxb = nx_ref[...]
    xwI = (jnp.dot(x1b, wia_ref[...], preferred_element_type=jnp.float32,
                   precision=_PREC_HI)
           + jnp.dot(nxb, wib_ref[...], preferred_element_type=jnp.float32,
                     precision=_PREC_HI))
    xwV = (jnp.dot(x1b, wva_ref[...], preferred_element_type=jnp.float32,
                   precision=_PREC_HI)
           + jnp.dot(nxb, wvb_ref[...], preferred_element_type=jnp.float32,
                     precision=_PREC_HI))
    yI_ref[...] = dinv * xwI
    yV_ref[...] = dinv * xwV


def _tc_prep(degp, x1, node_x, wia, wib, wva, wvb):
    f32 = jnp.float32
    full = lambda a, b: pl.BlockSpec((a, b), lambda i: (0, 0))
    return pl.pallas_call(
        _prep_body,
        grid=(N // NB,),
        in_specs=[
            pl.BlockSpec((1, NC * NS, NB), lambda i: (i, 0, 0)),
            pl.BlockSpec((NB, HID), lambda i: (i, 0)),
            pl.BlockSpec((NB, FEA), lambda i: (i, 0)),
            full(HID, HID), full(FEA, HID), full(HID, HID), full(FEA, HID),
        ],
        out_specs=[pl.BlockSpec((NB, HID), lambda i: (i, 0))] * 3,
        out_shape=[jax.ShapeDtypeStruct((N, HID), f32)] * 3,
    )(degp, x1, node_x, wia, wib, wva, wvb)


def _mid_body(sI_ref, sV_ref, yI_ref, yV_ref, dinv_ref, bI_ref, bV_ref,
              wI2_ref, wV2_ref, y2I_ref, y2V_ref):
    dinv = dinv_ref[...]
    tI = dinv * (sI_ref[...] + yI_ref[...]) + bI_ref[...]
    tV = dinv * (sV_ref[...] + yV_ref[...]) + bV_ref[...]
    y2I_ref[...] = dinv * jnp.dot(tI, wI2_ref[...],
                                  preferred_element_type=jnp.float32,
                                  precision=_PREC_HI)
    y2V_ref[...] = dinv * jnp.dot(tV, wV2_ref[...],
                                  preferred_element_type=jnp.float32,
                                  precision=_PREC_HI)


def _tc_mid(sI, sV, yI, yV, dinvb, bI1, bV1, wI2, wV2):
    f32 = jnp.float32
    blk = pl.BlockSpec((NB, HID), lambda i: (i, 0))
    full = lambda a, b: pl.BlockSpec((a, b), lambda i: (0, 0))
    return pl.pallas_call(
        _mid_body,
        grid=(N // NB,),
        in_specs=[blk, blk, blk, blk, blk,
                  full(1, HID), full(1, HID), full(HID, HID), full(HID, HID)],
        out_specs=[blk] * 2,
        out_shape=[jax.ShapeDtypeStruct((N, HID), f32)] * 2,
    )(sI, sV, yI, yV, dinvb, bI1, bV1, wI2, wV2)


def _fin_body(s2I_ref, s2V_ref, y2I_ref, y2V_ref, dinv_ref, bI_ref, bV_ref,
              wlI_ref, wlV_ref, bl_ref,
              x2I_ref, x2V_ref, pred_ref, su_ref, v_ref):
    dinv = dinv_ref[...]
    x2I = dinv * (s2I_ref[...] + y2I_ref[...]) + bI_ref[...]
    x2V = dinv * (s2V_ref[...] + y2V_ref[...]) + bV_ref[...]
    x2I_ref[...] = x2I
    x2V_ref[...] = x2V
    u = jnp.dot(x2I, wlI_ref[...], preferred_element_type=jnp.float32,
                precision=_PREC_HI)
    v = jnp.dot(x2V, wlV_ref[...], preferred_element_type=jnp.float32,
                precision=_PREC_HI)
    bl = bl_ref[...]
    pred_ref[...] = jnp.maximum(u + v + bl, 0.0)
    su_ref[...] = u + bl
    v_ref[...] = v


def _tc_fin(s2I, s2V, y2I, y2V, dinvb, bI2, bV2, wlI, wlV, blin):
    f32 = jnp.float32
    blk = pl.BlockSpec((NB, HID), lambda i: (i, 0))
    col = pl.BlockSpec((NB, 1), lambda i: (i, 0))
    full = lambda a, b: pl.BlockSpec((a, b), lambda i: (0, 0))
    return pl.pallas_call(
        _fin_body,
        grid=(N // NB,),
        in_specs=[blk, blk, blk, blk, blk,
                  full(1, HID), full(1, HID),
                  full(HID, 1), full(HID, 1), full(1, 1)],
        out_specs=[blk, blk, col, col, col],
        out_shape=[jax.ShapeDtypeStruct((N, HID), f32)] * 2
        + [jax.ShapeDtypeStruct((N, 1), f32)] * 3,
    )(s2I, s2V, y2I, y2V, dinvb, bI2, bV2, wlI, wlV, blin)


def _sc_deg_emul(col, ew):
    d0 = jnp.zeros((N,), jnp.float32).at[col].add(ew)
    out = jnp.zeros((NC * NS, 1, N), jnp.float32)
    return out.at[0, 0, :].set(d0)


def _sc_propagate_emul(yI, yV, row, col, ew):
    sI = jnp.zeros_like(yI).at[col].add(ew[:, None] * yI[row])
    sV = jnp.zeros_like(yV).at[col].add(ew[:, None] * yV[row])
    return sI, sV


def _sc_vstar_emul(su, v, perm):
    return jnp.maximum(su + v[perm], 0.0)


_SC_DEG = _sc_deg
_SC_PROP = _sc_propagate
_SC_VSTAR = _sc_vstar


def kernel(node_x, node_yx, edge_index, edge_weight,
           W_ih, W_hh, b_ih, b_hh, W_t, W_fc, b_fc,
           W_I1, b_I1, W_I2, b_I2, W_V1, b_V1, W_V2, b_V2,
           W_lin, b_lin, perm):
    row = edge_index[0]
    col = edge_index[1]
    perm32 = perm.astype(jnp.int32)

    degp = _SC_DEG(col, edge_weight).reshape(NC * NS, N // NB, NB)
    degp = degp.transpose(1, 0, 2)
    x1 = _tc_lstm(node_yx, W_ih.T, W_hh.T, (b_ih + b_hh)[None, :],
                  W_t, W_fc, b_fc[None, :])
    dinvb, yI, yV = _tc_prep(degp, x1, node_x,
                             W_I1[:HID], W_I1[HID:], W_V1[:HID], W_V1[HID:])
    sI, sV = _SC_PROP(yI, yV, row, col, edge_weight)
    y2I, y2V = _tc_mid(sI, sV, yI, yV, dinvb,
                       b_I1[None, :], b_V1[None, :], W_I2, W_V2)
    s2I, s2V = _SC_PROP(y2I, y2V, row, col, edge_weight)
    x2I, x2V, pred, su, v = _tc_fin(s2I, s2V, y2I, y2V, dinvb,
                                    b_I2[None, :], b_V2[None, :],
                                    W_lin[:HID], W_lin[HID:],
                                    b_lin[None, :])
    pred_star = _SC_VSTAR(su.reshape(N), v.reshape(N), perm32)
    return (pred.reshape(N), pred_star, x2I, x2V)


# speedup vs baseline: 5.9868x; 1.2831x over previous
import dataclasses
import functools

import jax
import jax.numpy as jnp
from jax import lax
from jax.experimental import pallas as pl
from jax.experimental.pallas import tpu as pltpu
from jax.experimental.pallas import tpu_sc as plsc

N = 10000
E = 320000
LAG = 50
FEA = 4
HID = 128
G4 = 4 * HID

NB = 1000
NC, NS = 2, 16
CH = 80
RPT = N // NS
RSTRIDE = 624
RSPAN = 640

_PREC_HI = lax.Precision.HIGHEST
_PREC_LSTM = lax.Precision.DEFAULT

@functools.cache
def _vec_mesh():
    return plsc.VectorSubcoreMesh(core_axis_name="c", subcore_axis_name="s",
                                  num_cores=NC, num_subcores=NS)


def _sc_params():
    cp = pltpu.CompilerParams()
    if "needs_layout_passes" in pltpu.CompilerParams.__dataclass_fields__:
        cp = dataclasses.replace(cp, needs_layout_passes=False)
    return cp


def _lstm_body(yx_ref, wih_ref, whhT_ref, b_ref, wt_ref, wfc_ref, bfc_ref,
               out_ref, h_ref, c_ref, acc_ref, den_ref):
    h_ref[...] = jnp.zeros_like(h_ref)
    c_ref[...] = jnp.zeros_like(c_ref)
    acc_ref[...] = jnp.zeros_like(acc_ref)
    den_ref[...] = jnp.zeros_like(den_ref)
    yx = yx_ref[...]
    wih = wih_ref[...]
    whhT = whhT_ref[...]
    b = b_ref[...]
    wt_all = wt_ref[...]

    def step(t, carry):
        oh_col = (lax.broadcasted_iota(jnp.int32, (LAG, 1), 0) == t)
        xt = jnp.dot(yx, oh_col.astype(jnp.float32),
                     preferred_element_type=jnp.float32)
        gates = (jnp.dot(h_ref[...], whhT,
                         preferred_element_type=jnp.float32,
                         precision=_PREC_LSTM)
                 + xt * wih + b)
        i = jax.nn.sigmoid(gates[:, :HID])
        f = jax.nn.sigmoid(gates[:, HID:2 * HID])
        g = jnp.tanh(gates[:, 2 * HID:3 * HID])
        o = jax.nn.sigmoid(gates[:, 3 * HID:])
        c = f * c_ref[...] + i * g
        hn = o * jnp.tanh(c)
        oh_row = (lax.broadcasted_iota(jnp.int32, (1, LAG), 1) == t)
        wt = jnp.dot(oh_row.astype(jnp.float32), wt_all,
                     preferred_element_type=jnp.float32)
        s = jnp.sum(hn * wt, axis=1, keepdims=True)
        e = jnp.exp(s)
        acc_ref[...] = acc_ref[...] + e * hn
        den_ref[...] = den_ref[...] + jnp.broadcast_to(e, den_ref.shape)
        h_ref[...] = hn
        c_ref[...] = c
        return carry

    lax.fori_loop(0, LAG, step, 0)
    att = acc_ref[...] / den_ref[...]
    out_ref[...] = jnp.maximum(
        jnp.dot(att, wfc_ref[...], preferred_element_type=jnp.float32,
                precision=_PREC_HI) + bfc_ref[...], 0.0)


def _tc_lstm(node_yx, wih_row, whhT, bsum, W_t, W_fc, bfc_row):
    f32 = jnp.float32
    return pl.pallas_call(
        _lstm_body,
        grid=(N // NB,),
        in_specs=[
            pl.BlockSpec((NB, LAG), lambda i: (i, 0)),
            pl.BlockSpec((1, G4), lambda i: (0, 0)),
            pl.BlockSpec((HID, G4), lambda i: (0, 0)),
            pl.BlockSpec((1, G4), lambda i: (0, 0)),
            pl.BlockSpec((LAG, HID), lambda i: (0, 0)),
            pl.BlockSpec((HID, HID), lambda i: (0, 0)),
            pl.BlockSpec((1, HID), lambda i: (0, 0)),
        ],
        out_specs=pl.BlockSpec((NB, HID), lambda i: (i, 0)),
        out_shape=jax.ShapeDtypeStruct((N, HID), f32),
        scratch_shapes=[pltpu.VMEM((NB, HID), f32)] * 4,
    )(node_yx, wih_row, whhT, bsum, W_t, W_fc, bfc_row)


def _sc_deg(col, ew):
    f32 = jnp.float32
    NW = NC * NS
    EPW = E // NW
    CHD = 2000
    NCHUNK = EPW // CHD

    @functools.partial(
        pl.kernel,
        out_type=jax.ShapeDtypeStruct((NW, 1, N), f32),
        mesh=_vec_mesh(),
        compiler_params=_sc_params(),
        scratch_types=[
            pltpu.VMEM((N,), f32),
            pltpu.VMEM((CHD,), jnp.int32),
            pltpu.VMEM((CHD,), f32),
        ],
    )
    def k(col_hbm, ew_hbm, out_hbm, dacc, colv, ewv):
        c = lax.axis_index("c")
        s = lax.axis_index("s")
        wid = s * NC + c
        zero16 = jnp.zeros((16,), f32)

        @pl.loop(0, N, step=16)
        def _(i):
            dacc[pl.ds(i, 16)] = zero16

        base = wid * EPW

        @pl.loop(0, NCHUNK)
        def _(kk):
            off = base + kk * CHD
            pltpu.sync_copy(col_hbm.at[pl.ds(off, CHD)], colv)
            pltpu.sync_copy(ew_hbm.at[pl.ds(off, CHD)], ewv)

            @pl.loop(0, CHD, step=16)
            def _(i):
                plsc.addupdate_scatter(dacc, [colv[pl.ds(i, 16)]],
                                       ewv[pl.ds(i, 16)])

        pltpu.sync_copy(dacc, out_hbm.at[wid].at[0])

    return k(col, ew)


def _sc_propagate(yI, yV, row, col, ew):
    f32 = jnp.float32
    EPT = E // NS
    NCHUNK = EPT // CH

    @functools.partial(
        pl.kernel,
        out_type=(jax.ShapeDtypeStruct((N, HID), f32),
                  jax.ShapeDtypeStruct((N, HID), f32)),
        mesh=_vec_mesh(),
        compiler_params=_sc_params(),
        scratch_types=[
            pltpu.VMEM((CH,), jnp.int32),
            pltpu.VMEM((1, CH), jnp.int32),
            pltpu.VMEM((CH,), f32),
            pltpu.VMEM((CH, HID), f32),
            pltpu.VMEM((128, HID), f32),
            pltpu.VMEM_SHARED((N, HID), f32),
            pltpu.SemaphoreType.DMA,
        ],
    )
    def k(yI_hbm, yV_hbm, row_hbm, col_hbm, ew_hbm, outI_hbm, outV_hbm,
          rowv, colv, ewv, gbuf, zbuf, acc, sem):
        c = lax.axis_index("c")
        s = lax.axis_index("s")
        zero16 = jnp.zeros((16,), f32)

        @pl.loop(0, 128)
        def _(i):
            for j in range(HID // 16):
                zbuf[i, pl.ds(j * 16, 16)] = zero16

        @pl.loop(0, 5)
        def _(j):
            pltpu.sync_copy(zbuf, acc.at[pl.ds(s * RSTRIDE + j * 128, 128)])

        plsc.subcore_barrier()
        base = s * EPT

        @pl.loop(0, NCHUNK)
        def _(kk):
            off = base + kk * CH
            pltpu.sync_copy(row_hbm.at[pl.ds(off, CH)], rowv)
            pltpu.sync_copy(col_hbm.at[pl.ds(off, CH)], colv.at[0])
            pltpu.sync_copy(ew_hbm.at[pl.ds(off, CH)], ewv)

            @pl.when(c == 0)
            def _():
                pltpu.async_copy(yI_hbm.at[rowv], gbuf, sem).wait()

            @pl.when(c == 1)
            def _():
                pltpu.async_copy(yV_hbm.at[rowv], gbuf, sem).wait()

            @pl.loop(0, CH, step=16)
            def _(i):
                ew16 = ewv[pl.ds(i, 16)]
                for l in range(16):
                    cv = lax.broadcast_in_dim(ew16[l], (16,), ())
                    for j in range(HID // 16):
                        sl = pl.ds(j * 16, 16)
                        gbuf[i + l, sl] = gbuf[i + l, sl] * cv

            pltpu.sync_copy(gbuf, acc.at[colv.at[0]], add=True)

        plsc.subcore_barrier()

        @pl.when(c == 0)
        def _():
            pltpu.sync_copy(acc.at[pl.ds(s * RSTRIDE, RSPAN)],
                            outI_hbm.at[pl.ds(s * RSTRIDE, RSPAN)])

        @pl.when(c == 1)
        def _():
            pltpu.sync_copy(acc.at[pl.ds(s * RSTRIDE, RSPAN)],
                            outV_hbm.at[pl.ds(s * RSTRIDE, RSPAN)])

    return k(yI, yV, row, col, ew)


def _sc_vstar(su, v, perm):
    f32 = jnp.float32
    TPT = 400

    @functools.partial(
        pl.kernel,
        out_type=jax.ShapeDtypeStruct((N,), f32),
        mesh=_vec_mesh(),
        compiler_params=_sc_params(),
        scratch_types=[
            pltpu.VMEM((N,), f32),
            pltpu.VMEM((TPT,), f32),
            pltpu.VMEM((TPT,), jnp.int32),
            pltpu.VMEM((TPT,), f32),
        ],
    )
    def k(su_hbm, v_hbm, perm_hbm, out_hbm, vv, suv, pv, ov):
        c = lax.axis_index("c")
        s = lax.axis_index("s")
        wid = s * NC + c

        @pl.when(wid < N // TPT)
        def _():
            base = wid * TPT
            pltpu.sync_copy(v_hbm, vv)
            pltpu.sync_copy(su_hbm.at[pl.ds(base, TPT)], suv)
            pltpu.sync_copy(perm_hbm.at[pl.ds(base, TPT)], pv)

            @pl.loop(0, TPT, step=16)
            def _(i):
                idx16 = pv[pl.ds(i, 16)]
                vg = plsc.load_gather(vv, [idx16])
                ov[pl.ds(i, 16)] = jnp.maximum(suv[pl.ds(i, 16)] + vg, 0.0)

            pltpu.sync_copy(ov, out_hbm.at[pl.ds(base, TPT)])

    return k(su, v, perm)


def _prep_body(dp_ref, x1_ref, nx_ref, wia_ref, wib_ref, wva_ref, wvb_ref,
               dinv_ref, yI_ref, yV_ref):
    dp = dp_ref[...]
    deg = jnp.sum(dp[0], axis=0) + 1.0
    dinv = lax.rsqrt(deg)[:, None]
    dinv_ref[...] = jnp.broadcast_to(dinv, dinv_ref.shape)
    x1b = x1_ref[...]
    nxb = nx_ref[...]
    xwI = (jnp.dot(x1b, wia_ref[...], preferred_element_type=jnp.float32,
                   precision=_PREC_HI)
           + jnp.dot(nxb, wib_ref[...], preferred_element_type=jnp.float32,
                     precision=_PREC_HI))
    xwV = (jnp.dot(x1b, wva_ref[...], preferred_element_type=jnp.float32,
                   precision=_PREC_HI)
           + jnp.dot(nxb, wvb_ref[...], preferred_element_type=jnp.float32,
                     precision=_PREC_HI))
    yI_ref[...] = dinv * xwI
    yV_ref[...] = dinv * xwV


def _tc_prep(degp, x1, node_x, wia, wib, wva, wvb):
    f32 = jnp.float32
    full = lambda a, b: pl.BlockSpec((a, b), lambda i: (0, 0))
    return pl.pallas_call(
        _prep_body,
        grid=(N // NB,),
        in_specs=[
            pl.BlockSpec((1, NC * NS, NB), lambda i: (i, 0, 0)),
            pl.BlockSpec((NB, HID), lambda i: (i, 0)),
            pl.BlockSpec((NB, FEA), lambda i: (i, 0)),
            full(HID, HID), full(FEA, HID), full(HID, HID), full(FEA, HID),
        ],
        out_specs=[pl.BlockSpec((NB, HID), lambda i: (i, 0))] * 3,
        out_shape=[jax.ShapeDtypeStruct((N, HID), f32)] * 3,
    )(degp, x1, node_x, wia, wib, wva, wvb)


def _mid_body(sI_ref, sV_ref, yI_ref, yV_ref, dinv_ref, bI_ref, bV_ref,
              wI2_ref, wV2_ref, y2I_ref, y2V_ref):
    dinv = dinv_ref[...]
    tI = dinv * (sI_ref[...] + yI_ref[...]) + bI_ref[...]
    tV = dinv * (sV_ref[...] + yV_ref[...]) + bV_ref[...]
    y2I_ref[...] = dinv * jnp.dot(tI, wI2_ref[...],
                                  preferred_element_type=jnp.float32,
                                  precision=_PREC_HI)
    y2V_ref[...] = dinv * jnp.dot(tV, wV2_ref[...],
                                  preferred_element_type=jnp.float32,
                                  precision=_PREC_HI)


def _tc_mid(sI, sV, yI, yV, dinvb, bI1, bV1, wI2, wV2):
    f32 = jnp.float32
    blk = pl.BlockSpec((NB, HID), lambda i: (i, 0))
    full = lambda a, b: pl.BlockSpec((a, b), lambda i: (0, 0))
    return pl.pallas_call(
        _mid_body,
        grid=(N // NB,),
        in_specs=[blk, blk, blk, blk, blk,
                  full(1, HID), full(1, HID), full(HID, HID), full(HID, HID)],
        out_specs=[blk] * 2,
        out_shape=[jax.ShapeDtypeStruct((N, HID), f32)] * 2,
    )(sI, sV, yI, yV, dinvb, bI1, bV1, wI2, wV2)


def _fin_body(s2I_ref, s2V_ref, y2I_ref, y2V_ref, dinv_ref, bI_ref, bV_ref,
              wlI_ref, wlV_ref, bl_ref,
              x2I_ref, x2V_ref, pred_ref, su_ref, v_ref):
    dinv = dinv_ref[...]
    x2I = dinv * (s2I_ref[...] + y2I_ref[...]) + bI_ref[...]
    x2V = dinv * (s2V_ref[...] + y2V_ref[...]) + bV_ref[...]
    x2I_ref[...] = x2I
    x2V_ref[...] = x2V
    u = jnp.dot(x2I, wlI_ref[...], preferred_element_type=jnp.float32,
                precision=_PREC_HI)
    v = jnp.dot(x2V, wlV_ref[...], preferred_element_type=jnp.float32,
                precision=_PREC_HI)
    bl = bl_ref[...]
    pred_ref[...] = jnp.maximum(u + v + bl, 0.0)
    su_ref[...] = u + bl
    v_ref[...] = v


def _tc_fin(s2I, s2V, y2I, y2V, dinvb, bI2, bV2, wlI, wlV, blin):
    f32 = jnp.float32
    blk = pl.BlockSpec((NB, HID), lambda i: (i, 0))
    col = pl.BlockSpec((NB, 1), lambda i: (i, 0))
    full = lambda a, b: pl.BlockSpec((a, b), lambda i: (0, 0))
    return pl.pallas_call(
        _fin_body,
        grid=(N // NB,),
        in_specs=[blk, blk, blk, blk, blk,
                  full(1, HID), full(1, HID),
                  full(HID, 1), full(HID, 1), full(1, 1)],
        out_specs=[blk, blk, col, col, col],
        out_shape=[jax.ShapeDtypeStruct((N, HID), f32)] * 2
        + [jax.ShapeDtypeStruct((N, 1), f32)] * 3,
    )(s2I, s2V, y2I, y2V, dinvb, bI2, bV2, wlI, wlV, blin)


def _sc_deg_emul(col, ew):
    d0 = jnp.zeros((N,), jnp.float32).at[col].add(ew)
    out = jnp.zeros((NC * NS, 1, N), jnp.float32)
    return out.at[0, 0, :].set(d0)


def _sc_propagate_emul(yI, yV, row, col, ew):
    sI = jnp.zeros_like(yI).at[col].add(ew[:, None] * yI[row])
    sV = jnp.zeros_like(yV).at[col].add(ew[:, None] * yV[row])
    return sI, sV


def _sc_vstar_emul(su, v, perm):
    return jnp.maximum(su + v[perm], 0.0)


_SC_DEG = _sc_deg
_SC_PROP = _sc_propagate
_SC_VSTAR = _sc_vstar


def kernel(node_x, node_yx, edge_index, edge_weight,
           W_ih, W_hh, b_ih, b_hh, W_t, W_fc, b_fc,
           W_I1, b_I1, W_I2, b_I2, W_V1, b_V1, W_V2, b_V2,
           W_lin, b_lin, perm):
    row = edge_index[0]
    col = edge_index[1]
    perm32 = perm.astype(jnp.int32)

    degp = _SC_DEG(col, edge_weight).reshape(NC * NS, N // NB, NB)
    degp = degp.transpose(1, 0, 2)
    x1 = _tc_lstm(node_yx, W_ih.T, W_hh.T, (b_ih + b_hh)[None, :],
                  W_t, W_fc, b_fc[None, :])
    dinvb, yI, yV = _tc_prep(degp, x1, node_x,
                             W_I1[:HID], W_I1[HID:], W_V1[:HID], W_V1[HID:])
    sI, sV = _SC_PROP(yI, yV, row, col, edge_weight)
    y2I, y2V = _tc_mid(sI, sV, yI, yV, dinvb,
                       b_I1[None, :], b_V1[None, :], W_I2, W_V2)
    s2I, s2V = _SC_PROP(y2I, y2V, row, col, edge_weight)
    x2I, x2V, pred, su, v = _tc_fin(s2I, s2V, y2I, y2V, dinvb,
                                    b_I2[None, :], b_V2[None, :],
                                    W_lin[:HID], W_lin[HID:],
                                    b_lin[None, :])
    pred_star = _SC_VSTAR(su.reshape(N), v.reshape(N), perm32)
    return (pred.reshape(N), pred_star, x2I, x2V)


# speedup vs baseline: 7.7549x; 1.2953x over previous
import dataclasses
import functools

import jax
import jax.numpy as jnp
from jax import lax
from jax.experimental import pallas as pl
from jax.experimental.pallas import tpu as pltpu
from jax.experimental.pallas import tpu_sc as plsc

N = 10000
E = 320000
LAG = 50
FEA = 4
HID = 128
G4 = 4 * HID

NB = 1000
NC, NS = 2, 16
CH = 80
RPT = N // NS
RSTRIDE = 624
RSPAN = 640

_PREC_HI = lax.Precision.HIGHEST
_PREC_LSTM = lax.Precision.DEFAULT

@functools.cache
def _vec_mesh():
    return plsc.VectorSubcoreMesh(core_axis_name="c", subcore_axis_name="s",
                                  num_cores=NC, num_subcores=NS)


def _sc_params():
    cp = pltpu.CompilerParams()
    if "needs_layout_passes" in pltpu.CompilerParams.__dataclass_fields__:
        cp = dataclasses.replace(cp, needs_layout_passes=False)
    return cp


def _lstm_body(yx_ref, wih_ref, whhT_ref, b_ref, wt_ref, wfc_ref, bfc_ref,
               out_ref, h_ref, c_ref, acc_ref, den_ref):
    h_ref[...] = jnp.zeros_like(h_ref)
    c_ref[...] = jnp.zeros_like(c_ref)
    acc_ref[...] = jnp.zeros_like(acc_ref)
    den_ref[...] = jnp.zeros_like(den_ref)
    yx = yx_ref[...]
    wih = wih_ref[...]
    whhT = whhT_ref[...]
    b = b_ref[...]
    wt_all = wt_ref[...]

    def step(t, carry):
        oh_col = (lax.broadcasted_iota(jnp.int32, (LAG, 1), 0) == t)
        xt = jnp.dot(yx, oh_col.astype(jnp.float32),
                     preferred_element_type=jnp.float32)
        gates = (jnp.dot(h_ref[...], whhT,
                         preferred_element_type=jnp.float32,
                         precision=_PREC_LSTM)
                 + xt * wih + b)
        i = jax.nn.sigmoid(gates[:, :HID])
        f = jax.nn.sigmoid(gates[:, HID:2 * HID])
        g = jnp.tanh(gates[:, 2 * HID:3 * HID])
        o = jax.nn.sigmoid(gates[:, 3 * HID:])
        c = f * c_ref[...] + i * g
        hn = o * jnp.tanh(c)
        oh_row = (lax.broadcasted_iota(jnp.int32, (1, LAG), 1) == t)
        wt = jnp.dot(oh_row.astype(jnp.float32), wt_all,
                     preferred_element_type=jnp.float32)
        s = jnp.sum(hn * wt, axis=1, keepdims=True)
        e = jnp.exp(s)
        acc_ref[...] = acc_ref[...] + e * hn
        den_ref[...] = den_ref[...] + jnp.broadcast_to(e, den_ref.shape)
        h_ref[...] = hn
        c_ref[...] = c
        return carry

    lax.fori_loop(0, LAG, step, 0)
    att = acc_ref[...] / den_ref[...]
    out_ref[...] = jnp.maximum(
        jnp.dot(att, wfc_ref[...], preferred_element_type=jnp.float32,
                precision=_PREC_HI) + bfc_ref[...], 0.0)


def _tc_lstm(node_yx, wih_row, whhT, bsum, W_t, W_fc, bfc_row):
    f32 = jnp.float32
    return pl.pallas_call(
        _lstm_body,
        grid=(N // NB,),
        in_specs=[
            pl.BlockSpec((NB, LAG), lambda i: (i, 0)),
            pl.BlockSpec((1, G4), lambda i: (0, 0)),
            pl.BlockSpec((HID, G4), lambda i: (0, 0)),
            pl.BlockSpec((1, G4), lambda i: (0, 0)),
            pl.BlockSpec((LAG, HID), lambda i: (0, 0)),
            pl.BlockSpec((HID, HID), lambda i: (0, 0)),
            pl.BlockSpec((1, HID), lambda i: (0, 0)),
        ],
        out_specs=pl.BlockSpec((NB, HID), lambda i: (i, 0)),
        out_shape=jax.ShapeDtypeStruct((N, HID), f32),
        scratch_shapes=[pltpu.VMEM((NB, HID), f32)] * 4,
    )(node_yx, wih_row, whhT, bsum, W_t, W_fc, bfc_row)


def _sc_deg(col, ew):
    f32 = jnp.float32
    NW = NC * NS
    EPW = E // NW
    CHD = 2000
    NCHUNK = EPW // CHD

    @functools.partial(
        pl.kernel,
        out_type=jax.ShapeDtypeStruct((NW, 1, N), f32),
        mesh=_vec_mesh(),
        compiler_params=_sc_params(),
        scratch_types=[
            pltpu.VMEM((N,), f32),
            pltpu.VMEM((CHD,), jnp.int32),
            pltpu.VMEM((CHD,), f32),
        ],
    )
    def k(col_hbm, ew_hbm, out_hbm, dacc, colv, ewv):
        c = lax.axis_index("c")
        s = lax.axis_index("s")
        wid = s * NC + c
        zero16 = jnp.zeros((16,), f32)

        @pl.loop(0, N, step=16)
        def _(i):
            dacc[pl.ds(i, 16)] = zero16

        base = wid * EPW

        @pl.loop(0, NCHUNK)
        def _(kk):
            off = base + kk * CHD
            pltpu.sync_copy(col_hbm.at[pl.ds(off, CHD)], colv)
            pltpu.sync_copy(ew_hbm.at[pl.ds(off, CHD)], ewv)

            @pl.loop(0, CHD, step=16)
            def _(i):
                plsc.addupdate_scatter(dacc, [colv[pl.ds(i, 16)]],
                                       ewv[pl.ds(i, 16)])

        pltpu.sync_copy(dacc, out_hbm.at[wid].at[0])

    return k(col, ew)


def _sc_propagate(yI, yV, row, col, ew):
    f32 = jnp.float32
    EPT = E // NS
    NCHUNK = EPT // CH

    @functools.partial(
        pl.kernel,
        out_type=(jax.ShapeDtypeStruct((N, HID), f32),
                  jax.ShapeDtypeStruct((N, HID), f32)),
        mesh=_vec_mesh(),
        compiler_params=_sc_params(),
        scratch_types=[
            pltpu.VMEM((2, CH), jnp.int32),
            pltpu.VMEM((2, CH), jnp.int32),
            pltpu.VMEM((2, CH), f32),
            pltpu.VMEM((2, CH, HID), f32),
            pltpu.VMEM((128, HID), f32),
            pltpu.VMEM_SHARED((N, HID), f32),
            pltpu.SemaphoreType.DMA,
            pltpu.SemaphoreType.DMA,
        ],
    )
    def k(yI_hbm, yV_hbm, row_hbm, col_hbm, ew_hbm, outI_hbm, outV_hbm,
          rowv, colv, ewv, gbuf, zbuf, acc, semA, semB):
        c = lax.axis_index("c")
        s = lax.axis_index("s")
        zero16 = jnp.zeros((16,), f32)
        sems = (semA, semB)

        @pl.loop(0, 128)
        def _(i):
            for j in range(HID // 16):
                zbuf[i, pl.ds(j * 16, 16)] = zero16

        @pl.loop(0, 5)
        def _(j):
            pltpu.sync_copy(zbuf, acc.at[pl.ds(s * RSTRIDE + j * 128, 128)])

        plsc.subcore_barrier()
        base = s * EPT

        def idx_load(kk, buf):
            off = base + kk * CH
            pltpu.sync_copy(row_hbm.at[pl.ds(off, CH)], rowv.at[buf])
            pltpu.sync_copy(col_hbm.at[pl.ds(off, CH)], colv.at[buf])
            pltpu.sync_copy(ew_hbm.at[pl.ds(off, CH)], ewv.at[buf])

        def gather_start(buf):
            @pl.when(c == 0)
            def _():
                pltpu.make_async_copy(yI_hbm.at[rowv.at[buf]], gbuf.at[buf],
                                      sems[buf]).start()

            @pl.when(c == 1)
            def _():
                pltpu.make_async_copy(yV_hbm.at[rowv.at[buf]], gbuf.at[buf],
                                      sems[buf]).start()

        def gather_wait(buf):
            pltpu.make_async_copy(yI_hbm.at[rowv.at[buf]], gbuf.at[buf],
                                  sems[buf]).wait()

        def scale_scatter(buf):
            @pl.loop(0, CH, step=16)
            def _(i):
                ew16 = ewv[buf, pl.ds(i, 16)]
                for l in range(16):
                    cv = lax.broadcast_in_dim(ew16[l], (16,), ())
                    for j in range(HID // 16):
                        sl = pl.ds(j * 16, 16)
                        gbuf[buf, i + l, sl] = gbuf[buf, i + l, sl] * cv

            pltpu.sync_copy(gbuf.at[buf], acc.at[colv.at[buf]], add=True)

        idx_load(0, 0)
        gather_start(0)

        @pl.loop(0, NCHUNK // 2)
        def _(m):
            k1 = 2 * m + 1
            k2 = 2 * m + 2
            idx_load(k1, 1)
            gather_start(1)
            gather_wait(0)
            scale_scatter(0)

            @pl.when(k2 < NCHUNK)
            def _():
                idx_load(k2, 0)
                gather_start(0)

            gather_wait(1)
            scale_scatter(1)

        plsc.subcore_barrier()

        @pl.when(c == 0)
        def _():
            pltpu.sync_copy(acc.at[pl.ds(s * RSTRIDE, RSPAN)],
                            outI_hbm.at[pl.ds(s * RSTRIDE, RSPAN)])

        @pl.when(c == 1)
        def _():
            pltpu.sync_copy(acc.at[pl.ds(s * RSTRIDE, RSPAN)],
                            outV_hbm.at[pl.ds(s * RSTRIDE, RSPAN)])

    return k(yI, yV, row, col, ew)


def _sc_vstar(su, v, perm):
    f32 = jnp.float32
    TPT = 400

    @functools.partial(
        pl.kernel,
        out_type=jax.ShapeDtypeStruct((N,), f32),
        mesh=_vec_mesh(),
        compiler_params=_sc_params(),
        scratch_types=[
            pltpu.VMEM((N,), f32),
            pltpu.VMEM((TPT,), f32),
            pltpu.VMEM((TPT,), jnp.int32),
            pltpu.VMEM((TPT,), f32),
        ],
    )
    def k(su_hbm, v_hbm, perm_hbm, out_hbm, vv, suv, pv, ov):
        c = lax.axis_index("c")
        s = lax.axis_index("s")
        wid = s * NC + c

        @pl.when(wid < N // TPT)
        def _():
            base = wid * TPT
            pltpu.sync_copy(v_hbm, vv)
            pltpu.sync_copy(su_hbm.at[pl.ds(base, TPT)], suv)
            pltpu.sync_copy(perm_hbm.at[pl.ds(base, TPT)], pv)

            @pl.loop(0, TPT, step=16)
            def _(i):
                idx16 = pv[pl.ds(i, 16)]
                vg = plsc.load_gather(vv, [idx16])
                ov[pl.ds(i, 16)] = jnp.maximum(suv[pl.ds(i, 16)] + vg, 0.0)

            pltpu.sync_copy(ov, out_hbm.at[pl.ds(base, TPT)])

    return k(su, v, perm)


def _prep_body(dp_ref, x1_ref, nx_ref, wia_ref, wib_ref, wva_ref, wvb_ref,
               dinv_ref, yI_ref, yV_ref):
    dp = dp_ref[...]
    deg = jnp.sum(dp[0], axis=0) + 1.0
    dinv = lax.rsqrt(deg)[:, None]
    dinv_ref[...] = jnp.broadcast_to(dinv, dinv_ref.shape)
    x1b = x1_ref[...]
    nxb = nx_ref[...]
    xwI = (jnp.dot(x1b, wia_ref[...], preferred_element_type=jnp.float32,
                   precision=_PREC_HI)
           + jnp.dot(nxb, wib_ref[...], preferred_element_type=jnp.float32,
                     precision=_PREC_HI))
    xwV = (jnp.dot(x1b, wva_ref[...], preferred_element_type=jnp.float32,
                   precision=_PREC_HI)
           + jnp.dot(nxb, wvb_ref[...], preferred_element_type=jnp.float32,
                     precision=_PREC_HI))
    yI_ref[...] = dinv * xwI
    yV_ref[...] = dinv * xwV


def _tc_prep(degp, x1, node_x, wia, wib, wva, wvb):
    f32 = jnp.float32
    full = lambda a, b: pl.BlockSpec((a, b), lambda i: (0, 0))
    return pl.pallas_call(
        _prep_body,
        grid=(N // NB,),
        in_specs=[
            pl.BlockSpec((1, NC * NS, NB), lambda i: (i, 0, 0)),
            pl.BlockSpec((NB, HID), lambda i: (i, 0)),
            pl.BlockSpec((NB, FEA), lambda i: (i, 0)),
            full(HID, HID), full(FEA, HID), full(HID, HID), full(FEA, HID),
        ],
        out_specs=[pl.BlockSpec((NB, HID), lambda i: (i, 0))] * 3,
        out_shape=[jax.ShapeDtypeStruct((N, HID), f32)] * 3,
    )(degp, x1, node_x, wia, wib, wva, wvb)


def _mid_body(sI_ref, sV_ref, yI_ref, yV_ref, dinv_ref, bI_ref, bV_ref,
              wI2_ref, wV2_ref, y2I_ref, y2V_ref):
    dinv = dinv_ref[...]
    tI = dinv * (sI_ref[...] + yI_ref[...]) + bI_ref[...]
    tV = dinv * (sV_ref[...] + yV_ref[...]) + bV_ref[...]
    y2I_ref[...] = dinv * jnp.dot(tI, wI2_ref[...],
                                  preferred_element_type=jnp.float32,
                                  precision=_PREC_HI)
    y2V_ref[...] = dinv * jnp.dot(tV, wV2_ref[...],
                                  preferred_element_type=jnp.float32,
                                  precision=_PREC_HI)


def _tc_mid(sI, sV, yI, yV, dinvb, bI1, bV1, wI2, wV2):
    f32 = jnp.float32
    blk = pl.BlockSpec((NB, HID), lambda i: (i, 0))
    full = lambda a, b: pl.BlockSpec((a, b), lambda i: (0, 0))
    return pl.pallas_call(
        _mid_body,
        grid=(N // NB,),
        in_specs=[blk, blk, blk, blk, blk,
                  full(1, HID), full(1, HID), full(HID, HID), full(HID, HID)],
        out_specs=[blk] * 2,
        out_shape=[jax.ShapeDtypeStruct((N, HID), f32)] * 2,
    )(sI, sV, yI, yV, dinvb, bI1, bV1, wI2, wV2)


def _fin_body(s2I_ref, s2V_ref, y2I_ref, y2V_ref, dinv_ref, bI_ref, bV_ref,
              wlI_ref, wlV_ref, bl_ref,
              x2I_ref, x2V_ref, pred_ref, su_ref, v_ref):
    dinv = dinv_ref[...]
    x2I = dinv * (s2I_ref[...] + y2I_ref[...]) + bI_ref[...]
    x2V = dinv * (s2V_ref[...] + y2V_ref[...]) + bV_ref[...]
    x2I_ref[...] = x2I
    x2V_ref[...] = x2V
    u = jnp.dot(x2I, wlI_ref[...], preferred_element_type=jnp.float32,
                precision=_PREC_HI)
    v = jnp.dot(x2V, wlV_ref[...], preferred_element_type=jnp.float32,
                precision=_PREC_HI)
    bl = bl_ref[...]
    pred_ref[...] = jnp.maximum(u + v + bl, 0.0)
    su_ref[...] = u + bl
    v_ref[...] = v


def _tc_fin(s2I, s2V, y2I, y2V, dinvb, bI2, bV2, wlI, wlV, blin):
    f32 = jnp.float32
    blk = pl.BlockSpec((NB, HID), lambda i: (i, 0))
    col = pl.BlockSpec((NB, 1), lambda i: (i, 0))
    full = lambda a, b: pl.BlockSpec((a, b), lambda i: (0, 0))
    return pl.pallas_call(
        _fin_body,
        grid=(N // NB,),
        in_specs=[blk, blk, blk, blk, blk,
                  full(1, HID), full(1, HID),
                  full(HID, 1), full(HID, 1), full(1, 1)],
        out_specs=[blk, blk, col, col, col],
        out_shape=[jax.ShapeDtypeStruct((N, HID), f32)] * 2
        + [jax.ShapeDtypeStruct((N, 1), f32)] * 3,
    )(s2I, s2V, y2I, y2V, dinvb, bI2, bV2, wlI, wlV, blin)


def _sc_deg_emul(col, ew):
    d0 = jnp.zeros((N,), jnp.float32).at[col].add(ew)
    out = jnp.zeros((NC * NS, 1, N), jnp.float32)
    return out.at[0, 0, :].set(d0)


def _sc_propagate_emul(yI, yV, row, col, ew):
    sI = jnp.zeros_like(yI).at[col].add(ew[:, None] * yI[row])
    sV = jnp.zeros_like(yV).at[col].add(ew[:, None] * yV[row])
    return sI, sV


def _sc_vstar_emul(su, v, perm):
    return jnp.maximum(su + v[perm], 0.0)


_SC_DEG = _sc_deg
_SC_PROP = _sc_propagate
_SC_VSTAR = _sc_vstar


def kernel(node_x, node_yx, edge_index, edge_weight,
           W_ih, W_hh, b_ih, b_hh, W_t, W_fc, b_fc,
           W_I1, b_I1, W_I2, b_I2, W_V1, b_V1, W_V2, b_V2,
           W_lin, b_lin, perm):
    row = edge_index[0]
    col = edge_index[1]
    perm32 = perm.astype(jnp.int32)

    degp = _SC_DEG(col, edge_weight).reshape(NC * NS, N // NB, NB)
    degp = degp.transpose(1, 0, 2)
    x1 = _tc_lstm(node_yx, W_ih.T, W_hh.T, (b_ih + b_hh)[None, :],
                  W_t, W_fc, b_fc[None, :])
    dinvb, yI, yV = _tc_prep(degp, x1, node_x,
                             W_I1[:HID], W_I1[HID:], W_V1[:HID], W_V1[HID:])
    sI, sV = _SC_PROP(yI, yV, row, col, edge_weight)
    y2I, y2V = _tc_mid(sI, sV, yI, yV, dinvb,
                       b_I1[None, :], b_V1[None, :], W_I2, W_V2)
    s2I, s2V = _SC_PROP(y2I, y2V, row, col, edge_weight)
    x2I, x2V, pred, su, v = _tc_fin(s2I, s2V, y2I, y2V, dinvb,
                                    b_I2[None, :], b_V2[None, :],
                                    W_lin[:HID], W_lin[HID:],
                                    b_lin[None, :])
    pred_star = _SC_VSTAR(su.reshape(N), v.reshape(N), perm32)
    return (pred.reshape(N), pred_star, x2I, x2V)
